# Initial kernel scaffold; baseline (speedup 1.0000x reference)
import jax
import jax.numpy as jnp
from jax.experimental import pallas as pl


def kernel(tokens, h0, c0, embedding_pad, wih_0, whh_0, bias_0, wih_1, whh_1, bias_1, fc_w_pad, fc_b_pad):
    raise NotImplementedError("write your pallas kernel here")


# speedup vs baseline: 1.4164x; 1.4164x over previous
import jax
import jax.numpy as jnp
from jax import lax
from jax.experimental import pallas as pl
from jax.experimental.pallas import tpu as pltpu


def _make_body(L, T, B, H, N):
    G = 4 * H

    def body(tok_ref, emb_hbm,
             wih0_ref, b0_ref, wf0_ref, wb0_ref,
             wih1_ref, b1_ref, wf1_ref, wb1_ref,
             h0_ref, c0_ref, fcw_ref, fcb_ref,
             hn_ref, cn_ref, sig_ref,
             x3_ref, gin_ref, act_ref, sem):
        for r in range(N):
            pltpu.make_async_copy(emb_hbm.at[tok_ref[r]], x3_ref.at[r],
                                  sem).start()

        lane = lax.broadcasted_iota(jnp.int32, (1, G), 1)
        scale = jnp.where((lane >= 2 * H) & (lane < 3 * H), 1.0, 0.5)

        def step(g, c):
            th = jnp.tanh(g * scale)
            i_g = th[:, 0:H] * 0.5 + 0.5
            f_g = th[:, H:2 * H] * 0.5 + 0.5
            g_g = th[:, 2 * H:3 * H]
            o_g = th[:, 3 * H:4 * H] * 0.5 + 0.5
            c = f_g * c + i_g * g_g
            return o_g * jnp.tanh(c), c

        pltpu.make_async_copy(x3_ref, x3_ref, sem).wait()

        x0 = x3_ref[...].reshape(N, emb_hbm.shape[2])
        gin_ref[...] = (jnp.dot(x0, wih0_ref[...],
                                preferred_element_type=jnp.float32)
                        + b0_ref[...])

        last_bwd = None
        for l in range(L):
            wf = (wf0_ref if l == 0 else wf1_ref)[...]
            wb = (wb0_ref if l == 0 else wb1_ref)[...]
            hf = h0_ref[2 * l]
            hb = h0_ref[2 * l + 1]
            cf = c0_ref[2 * l]
            cb = c0_ref[2 * l + 1]

            for t in range(T):
                rt = T - 1 - t
                gf = (gin_ref[t * B:(t + 1) * B, 0:G]
                      + jnp.dot(hf, wf, preferred_element_type=jnp.float32))
                gb = (gin_ref[rt * B:(rt + 1) * B, G:2 * G]
                      + jnp.dot(hb, wb, preferred_element_type=jnp.float32))
                hf, cf = step(gf, cf)
                hb, cb = step(gb, cb)
                if l < L - 1:
                    act_ref[t * B:(t + 1) * B, 0:H] = hf
                    act_ref[rt * B:(rt + 1) * B, H:2 * H] = hb
                if l == L - 1 and t == 0:
                    last_bwd = hb

            hn_ref[2 * l] = hf
            hn_ref[2 * l + 1] = hb
            cn_ref[2 * l] = cf
            cn_ref[2 * l + 1] = cb

            if l < L - 1:
                gin_ref[...] = (jnp.dot(act_ref[...], wih1_ref[...],
                                        preferred_element_type=jnp.float32)
                                + b1_ref[...])

        sig_ref[...] = 0.5 * jnp.tanh(
            0.5 * (jnp.dot(last_bwd, fcw_ref[...],
                           preferred_element_type=jnp.float32)
                   + fcb_ref[...])) + 0.5

    return body


def kernel(tokens, h0, c0, embedding_pad, wih_0, whh_0, bias_0,
           wih_1, whh_1, bias_1, fc_w_pad, fc_b_pad):
    B, T = tokens.shape
    H = whh_0.shape[0]
    G = 4 * H
    L = 2
    N = T * B
    vocab_p, EP = embedding_pad.shape
    OP = fc_w_pad.shape[1]

    tok = tokens.T.reshape(N).astype(jnp.int32)
    emb3 = embedding_pad.reshape(vocab_p, 1, EP)
    wf0, wb0 = whh_0[:, :G], whh_0[:, G:]
    wf1, wb1 = whh_1[:, :G], whh_1[:, G:]

    full = lambda shape: pl.BlockSpec(shape, lambda i, *_: (0,) * len(shape))
    grid_spec = pltpu.PrefetchScalarGridSpec(
        num_scalar_prefetch=1,
        grid=(1,),
        in_specs=[
            pl.BlockSpec(memory_space=pl.ANY),
            full(wih_0.shape), full(bias_0.shape), full(wf0.shape),
            full(wb0.shape),
            full(wih_1.shape), full(bias_1.shape), full(wf1.shape),
            full(wb1.shape),
            full(h0.shape), full(c0.shape),
            full(fc_w_pad.shape), full(fc_b_pad.shape),
        ],
        out_specs=(
            full((2 * L, B, H)),
            full((2 * L, B, H)),
            full((B, OP)),
        ),
        scratch_shapes=[
            pltpu.VMEM((N, 1, EP), jnp.float32),
            pltpu.VMEM((N, 2 * G), jnp.float32),
            pltpu.VMEM((N, 2 * H), jnp.float32),
            pltpu.SemaphoreType.DMA,
        ],
    )
    out_shape = (
        jax.ShapeDtypeStruct((2 * L, B, H), jnp.float32),
        jax.ShapeDtypeStruct((2 * L, B, H), jnp.float32),
        jax.ShapeDtypeStruct((B, OP), jnp.float32),
    )
    hn, cn, sig = pl.pallas_call(
        _make_body(L, T, B, H, N),
        out_shape=out_shape,
        grid_spec=grid_spec,
        compiler_params=pltpu.CompilerParams(
            dimension_semantics=("arbitrary",),
            disable_bounds_checks=True),
    )(tok, emb3, wih_0, bias_0, wf0, wb0, wih_1, bias_1, wf1, wb1,
      h0, c0, fc_w_pad, fc_b_pad)

    return sig[:, 0], (hn, cn)


# speedup vs baseline: 2.4394x; 1.7223x over previous
import jax
import jax.numpy as jnp
from jax import lax
from jax.experimental import pallas as pl
from jax.experimental.pallas import tpu as pltpu


def _make_body(L, T, B, H, N, EP):
    G = 4 * H
    G2 = 2 * G
    H2 = 2 * H

    def body(tok_ref, emb_hbm,
             wih0_ref, b0_ref, whh0_ref,
             wih1_ref, b1_ref, whh1_ref,
             h0_ref, c0_ref, fcw_ref, fcb_ref,
             hn_ref, cn_ref, p_ref,
             x3_ref, x2_ref, gin_ref, act_ref, sem):
        for r in range(N):
            src = tok_ref[(r % B) * T + (r // B)]
            pltpu.make_async_copy(emb_hbm.at[src], x3_ref.at[r], sem).start()

        f32 = jnp.float32
        wT = []
        for w_ref in (whh0_ref, whh1_ref):
            w = w_ref[...]
            wT.append((jnp.swapaxes(w[:, :G], 0, 1),
                       jnp.swapaxes(w[:, G:], 0, 1)))
        wih1T = jnp.swapaxes(wih1_ref[...], 0, 1)
        b1T = jnp.broadcast_to(
            jnp.swapaxes(b1_ref[...], 0, 1), (G2, B))
        fcw0T = jnp.swapaxes(fcw_ref[:, 0:1], 0, 1)
        fcb0 = fcb_ref[:, 0:1]
        hs = [jnp.swapaxes(h0_ref[i], 0, 1) for i in range(2 * L)]
        cs = [jnp.swapaxes(c0_ref[i], 0, 1) for i in range(2 * L)]

        row = lax.broadcasted_iota(jnp.int32, (G, B), 0)
        scale = jnp.where((row >= 2 * H) & (row < 3 * H), 1.0, 0.5)

        def step(g, c):
            th = jnp.tanh(g * scale)
            i_g = th[0:H] * 0.5 + 0.5
            f_g = th[H:2 * H] * 0.5 + 0.5
            g_g = th[2 * H:3 * H]
            o_g = th[3 * H:] * 0.5 + 0.5
            c = f_g * c + i_g * g_g
            return o_g * jnp.tanh(c), c

        pltpu.make_async_copy(x3_ref, x3_ref, sem).wait()
        x2_ref[...] = x3_ref[...].reshape(N, EP)

        last_bwd = None
        for l in range(L):
            if l == 0:
                for t in range(T):
                    blk = (jnp.dot(x2_ref[t * B:(t + 1) * B, :], wih0_ref[...],
                                   preferred_element_type=f32) + b0_ref[...])
                    gin_ref[t * G2:(t + 1) * G2, :] = jnp.swapaxes(blk, 0, 1)
            else:
                for t in range(T):
                    gin_ref[t * G2:(t + 1) * G2, :] = (
                        jnp.dot(wih1T, act_ref[t * H2:(t + 1) * H2, :],
                                preferred_element_type=f32) + b1T)

            wfT, wbT = wT[l]
            hf, cf = hs[2 * l], cs[2 * l]
            hb, cb = hs[2 * l + 1], cs[2 * l + 1]
            for t in range(T):
                rt = T - 1 - t
                gf = (gin_ref[t * G2:t * G2 + G, :]
                      + jnp.dot(wfT, hf, preferred_element_type=f32))
                gb = (gin_ref[rt * G2 + G:(rt + 1) * G2, :]
                      + jnp.dot(wbT, hb, preferred_element_type=f32))
                hf, cf = step(gf, cf)
                hb, cb = step(gb, cb)
                if l < L - 1:
                    act_ref[t * H2:t * H2 + H, :] = hf
                    act_ref[rt * H2 + H:(rt + 1) * H2, :] = hb
                if l == L - 1 and t == 0:
                    last_bwd = hb

            hn_ref[2 * l] = jnp.swapaxes(hf, 0, 1)
            hn_ref[2 * l + 1] = jnp.swapaxes(hb, 0, 1)
            cn_ref[2 * l] = jnp.swapaxes(cf, 0, 1)
            cn_ref[2 * l + 1] = jnp.swapaxes(cb, 0, 1)

        p_ref[...] = 0.5 * jnp.tanh(
            0.5 * (jnp.dot(fcw0T, last_bwd, preferred_element_type=f32)
                   + fcb0)) + 0.5

    return body


def kernel(tokens, h0, c0, embedding_pad, wih_0, whh_0, bias_0,
           wih_1, whh_1, bias_1, fc_w_pad, fc_b_pad):
    B, T = tokens.shape
    H = whh_0.shape[0]
    L = 2
    N = T * B
    vocab_p, EP = embedding_pad.shape

    tok = tokens.reshape(N).astype(jnp.int32)
    emb3 = embedding_pad.reshape(vocab_p, 1, EP)

    full = lambda shape: pl.BlockSpec(shape, lambda i, *_: (0,) * len(shape))
    grid_spec = pltpu.PrefetchScalarGridSpec(
        num_scalar_prefetch=1,
        grid=(1,),
        in_specs=[
            pl.BlockSpec(memory_space=pl.ANY),
            full(wih_0.shape), full(bias_0.shape), full(whh_0.shape),
            full(wih_1.shape), full(bias_1.shape), full(whh_1.shape),
            full(h0.shape), full(c0.shape),
            full(fc_w_pad.shape), full(fc_b_pad.shape),
        ],
        out_specs=(
            full((2 * L, B, H)),
            full((2 * L, B, H)),
            full((1, B)),
        ),
        scratch_shapes=[
            pltpu.VMEM((N, 1, EP), jnp.float32),
            pltpu.VMEM((N, EP), jnp.float32),
            pltpu.VMEM((T * 8 * H * 2, B), jnp.float32),
            pltpu.VMEM((T * 2 * H, B), jnp.float32),
            pltpu.SemaphoreType.DMA,
        ],
    )
    out_shape = (
        jax.ShapeDtypeStruct((2 * L, B, H), jnp.float32),
        jax.ShapeDtypeStruct((2 * L, B, H), jnp.float32),
        jax.ShapeDtypeStruct((1, B), jnp.float32),
    )
    hn, cn, p = pl.pallas_call(
        _make_body(L, T, B, H, N, EP),
        out_shape=out_shape,
        grid_spec=grid_spec,
        compiler_params=pltpu.CompilerParams(
            dimension_semantics=("arbitrary",),
            disable_bounds_checks=True),
    )(tok, emb3, wih_0, bias_0, whh_0, wih_1, bias_1, whh_1,
      h0, c0, fc_w_pad, fc_b_pad)

    return p.reshape(B), (hn, cn)
